# baseline (speedup 1.0000x reference)
import functools

import jax
import jax.numpy as jnp
from jax import lax
from jax.experimental import pallas as pl
from jax.experimental.pallas import tpu as pltpu
from jax.experimental.pallas import tpu_sc as plsc

NQ = 4
CHUNK = 128
LANES = 16


def _make_kernel(n, dim, nc, ns):
    nw = nc * ns
    assert n % (nw * CHUNK) == 0
    chunks_per_w = n // (nw * CHUNK)
    idx_per_w = chunks_per_w * CHUNK
    row_elems = CHUNK * dim * NQ

    mesh = plsc.VectorSubcoreMesh(core_axis_name="c", subcore_axis_name="s")

    @functools.partial(
        pl.kernel,
        out_type=jax.ShapeDtypeStruct((n * dim * NQ,), jnp.float32),
        mesh=mesh,
        compiler_params=pltpu.CompilerParams(needs_layout_passes=False,
                                             use_tc_tiling_on_sc=False),
        scratch_types=[
            pltpu.VMEM((idx_per_w,), jnp.int32),
            pltpu.VMEM((CHUNK, dim), jnp.float32),
            pltpu.VMEM((CHUNK, dim), jnp.float32),
            pltpu.VMEM((CHUNK, dim), jnp.float32),
            pltpu.VMEM((CHUNK, dim), jnp.float32),
            pltpu.VMEM((row_elems,), jnp.float32),
            pltpu.SemaphoreType.DMA,
        ],
    )
    def qembed(x_ref, s_ref, vi_ref, vj_ref, vk_ref, out_ref,
               idx_v, g0, g1, g2, g3, o_v, sem):
        wid = lax.axis_index("c") * ns + lax.axis_index("s")
        base = wid * idx_per_w
        pltpu.sync_copy(x_ref.at[pl.ds(base, idx_per_w)], idx_v)

        iota = lax.iota(jnp.int32, LANES)
        pats = [[iota * NQ + (h * LANES * NQ + q) for h in range(dim // LANES)]
                for q in range(NQ)]

        gbufs = (g0, g1, g2, g3)
        tables = (s_ref, vi_ref, vj_ref, vk_ref)

        def do_chunk(t, carry):
            idx_row = idx_v.at[pl.ds(t * CHUNK, CHUNK)]
            cps = [pltpu.async_copy(tables[q].at[idx_row], gbufs[q], sem)
                   for q in range(NQ)]
            for cp in cps:
                cp.wait()

            def interleave(i, carry2):
                m = i * (dim * NQ)
                for q in range(NQ):
                    for h in range(dim // LANES):
                        vals = gbufs[q][i, pl.ds(h * LANES, LANES)]
                        plsc.store_scatter(o_v, [pats[q][h] + m], vals)
                return carry2

            lax.fori_loop(0, CHUNK, interleave, 0, unroll=2)
            pltpu.sync_copy(o_v, out_ref.at[pl.ds((base + t * CHUNK) * dim * NQ,
                                                  row_elems)])
            return carry

        lax.fori_loop(0, chunks_per_w, do_chunk, 0)

    return qembed


@jax.jit
def kernel(x, scalar, vector_i, vector_j, vector_k):
    b, l = x.shape
    vocab, dim = scalar.shape
    n = b * l
    info = plsc.get_sparse_core_info()
    k = _make_kernel(n, dim, info.num_cores, info.num_subcores)
    out = k(x.reshape(n).astype(jnp.int32), scalar, vector_i, vector_j, vector_k)
    return out.reshape(b, l, dim, NQ)


# speedup vs baseline: 2.1483x; 2.1483x over previous
import functools

import jax
import jax.numpy as jnp
from jax import lax
from jax.experimental import pallas as pl
from jax.experimental.pallas import tpu as pltpu
from jax.experimental.pallas import tpu_sc as plsc

NQ = 4
CHUNK = 128
LANES = 16
PACK = 4


def _make_kernel(l_seq, b, dim, nc, ns):
    nw = nc * ns
    assert b % (nw * CHUNK) == 0 or b == nw * CHUNK
    row_elems = PACK * dim

    mesh = plsc.VectorSubcoreMesh(core_axis_name="c", subcore_axis_name="s")

    @functools.partial(
        pl.kernel,
        out_type=jax.ShapeDtypeStruct((l_seq, b, dim * NQ), jnp.float32),
        mesh=mesh,
        compiler_params=pltpu.CompilerParams(needs_layout_passes=False),
        scratch_types=[
            pltpu.VMEM((l_seq, CHUNK), jnp.int32),
            pltpu.VMEM((CHUNK,), jnp.int32),
            pltpu.VMEM((CHUNK, row_elems), jnp.float32),
            pltpu.VMEM((CHUNK, row_elems), jnp.float32),
            pltpu.VMEM((CHUNK, row_elems), jnp.float32),
            pltpu.VMEM((CHUNK, row_elems), jnp.float32),
            pltpu.VMEM((CHUNK, dim * NQ), jnp.float32),
            pltpu.SemaphoreType.DMA,
        ],
    )
    def qembed(xt_ref, s_ref, vi_ref, vj_ref, vk_ref, out_ref,
               idx_v, idxq_v, g0, g1, g2, g3, o_v, sem):
        wid = lax.axis_index("c") * ns + lax.axis_index("s")
        col0 = wid * CHUNK
        pltpu.sync_copy(xt_ref.at[:, pl.ds(col0, CHUNK)], idx_v)

        iota = lax.iota(jnp.int32, LANES)
        gbufs = (g0, g1, g2, g3)
        tables = (s_ref, vi_ref, vj_ref, vk_ref)
        nj = CHUNK // LANES

        def do_chunk(t, carry):
            for j in range(nj):
                vv = idx_v[t, pl.ds(j * LANES, LANES)]
                idxq_v[pl.ds(j * LANES, LANES)] = lax.shift_right_logical(vv, 2)
            cps = [pltpu.async_copy(tables[q].at[idxq_v], gbufs[q], sem)
                   for q in range(NQ)]
            for cp in cps:
                cp.wait()

            rows = [iota + j * LANES for j in range(nj)]
            subs = [(idx_v[t, pl.ds(j * LANES, LANES)] & 3) * dim
                    for j in range(nj)]

            def per_d(d, carry2):
                for j in range(nj):
                    for q in range(NQ):
                        vals = plsc.load_gather(gbufs[q], [rows[j], subs[j] + d])
                        plsc.store_scatter(
                            o_v, [rows[j], jnp.full((LANES,), 0, jnp.int32)
                                  + (d * NQ + q)], vals)
                return carry2

            lax.fori_loop(0, dim, per_d, 0)
            pltpu.sync_copy(o_v, out_ref.at[t, pl.ds(col0, CHUNK), :])
            return carry

        lax.fori_loop(0, l_seq, do_chunk, 0)

    return qembed


@jax.jit
def kernel(x, scalar, vector_i, vector_j, vector_k):
    b, l_seq = x.shape
    vocab, dim = scalar.shape
    info = plsc.get_sparse_core_info()
    k = _make_kernel(l_seq, b, dim, info.num_cores, info.num_subcores)
    tabs = [t.reshape(vocab // PACK, PACK * dim)
            for t in (scalar, vector_i, vector_j, vector_k)]
    out = k(x.T.astype(jnp.int32), *tabs)
    return out.reshape(l_seq, b, dim, NQ).transpose(1, 0, 2, 3)
